# Initial kernel scaffold; baseline (speedup 1.0000x reference)
import functools

import jax
import jax.numpy as jnp
from jax import lax
from jax.experimental import pallas as pl
from jax.experimental.pallas import tpu as pltpu
from jax.experimental.pallas import tpu_sc as plsc

N = 10000
E = 320000
D = 128

NC = 2
NS = 16
NW = NC * NS
CHUNK = 128
CPW = 79
EPW = CHUNK * CPW
E_PAD = EPW * NW
N_PAD = 10016
RPS = N_PAD // NS
DEGL = 16

ROWB = 1000

_mesh = plsc.VectorSubcoreMesh(core_axis_name="c", subcore_axis_name="s")


@functools.partial(
    pl.kernel,
    out_type=jax.ShapeDtypeStruct((NC, N_PAD, DEGL), jnp.float32),
    mesh=_mesh,
    scratch_types=[
        pltpu.VMEM((CHUNK,), jnp.int32),
        pltpu.VMEM((CHUNK, DEGL), jnp.float32),
        pltpu.VMEM_SHARED((N_PAD, DEGL), jnp.float32),
    ],
)
def _sc_degree(dst_hbm, out_hbm, didx, buf, acc):
    cid = lax.axis_index("c")
    sid = lax.axis_index("s")
    wid = cid * NS + sid

    @pl.loop(0, CHUNK)
    def _(i):
        buf[i, pl.ds(0, DEGL)] = jnp.zeros((DEGL,), jnp.float32)

    row0 = sid * RPS

    @pl.loop(0, 4)
    def _(b):
        pltpu.sync_copy(buf, acc.at[pl.ds(row0 + b * CHUNK, CHUNK)])

    pltpu.sync_copy(buf.at[pl.ds(0, RPS - 4 * CHUNK)],
                    acc.at[pl.ds(row0 + 4 * CHUNK, RPS - 4 * CHUNK)])

    @pl.loop(0, CHUNK)
    def _(i):
        buf[i, pl.ds(0, DEGL)] = jnp.ones((DEGL,), jnp.float32)

    plsc.subcore_barrier()

    base0 = wid * EPW

    @pl.loop(0, CPW)
    def _(c):
        pltpu.sync_copy(dst_hbm.at[pl.ds(base0 + c * CHUNK, CHUNK)], didx)
        pltpu.sync_copy(buf, acc.at[didx], add=True)

    plsc.subcore_barrier()
    pltpu.sync_copy(acc.at[pl.ds(row0, RPS)], out_hbm.at[cid, pl.ds(row0, RPS)])


@functools.partial(
    pl.kernel,
    out_type=jax.ShapeDtypeStruct((NC, N_PAD, D), jnp.float32),
    mesh=_mesh,
    scratch_types=[
        pltpu.VMEM((CHUNK,), jnp.int32),
        pltpu.VMEM((CHUNK,), jnp.int32),
        pltpu.VMEM((CHUNK, D), jnp.float32),
        pltpu.VMEM_SHARED((N_PAD, D), jnp.float32),
        pltpu.SemaphoreType.DMA,
    ],
)
def _sc_aggregate(table_hbm, src_hbm, dst_hbm, out_hbm, sidx, didx, rows, acc, sem):
    cid = lax.axis_index("c")
    sid = lax.axis_index("s")
    wid = cid * NS + sid

    @pl.loop(0, CHUNK)
    def _(i):
        @pl.loop(0, D // 16)
        def _(j):
            rows[i, pl.ds(j * 16, 16)] = jnp.zeros((16,), jnp.float32)

    row0 = sid * RPS

    @pl.loop(0, 4)
    def _(b):
        pltpu.sync_copy(rows, acc.at[pl.ds(row0 + b * CHUNK, CHUNK)])

    pltpu.sync_copy(rows.at[pl.ds(0, RPS - 4 * CHUNK)],
                    acc.at[pl.ds(row0 + 4 * CHUNK, RPS - 4 * CHUNK)])

    plsc.subcore_barrier()

    base0 = wid * EPW

    @pl.loop(0, CPW)
    def _(c):
        b = base0 + c * CHUNK
        pltpu.sync_copy(src_hbm.at[pl.ds(b, CHUNK)], sidx)
        pltpu.sync_copy(dst_hbm.at[pl.ds(b, CHUNK)], didx)
        pltpu.async_copy(table_hbm.at[sidx], rows, sem).wait()
        pltpu.sync_copy(rows, acc.at[didx], add=True)

    plsc.subcore_barrier()
    pltpu.sync_copy(acc.at[pl.ds(row0, RPS)], out_hbm.at[cid, pl.ds(row0, RPS)])


def _dinv_block(deg_ref):
    deg = deg_ref[0, :, 0:1] + deg_ref[1, :, 0:1] + 1.0
    return lax.rsqrt(deg)


def _tc_pre_body(x_ref, w_ref, deg_ref, o_ref):
    h = jnp.dot(x_ref[...], w_ref[...], preferred_element_type=jnp.float32)
    o_ref[...] = h * _dinv_block(deg_ref)


def _tc_mid_body(agg_ref, hs_ref, deg_ref, b_ref, w_ref, o_ref):
    dinv = _dinv_block(deg_ref)
    t = (agg_ref[0] + agg_ref[1] + hs_ref[...]) * dinv + b_ref[...]
    r = jnp.maximum(t, 0.0)
    o_ref[...] = jnp.dot(r, w_ref[...], preferred_element_type=jnp.float32) * dinv


def _tc_post_body(agg_ref, hs_ref, deg_ref, b_ref, o_ref):
    dinv = _dinv_block(deg_ref)
    t = (agg_ref[0] + agg_ref[1] + hs_ref[...]) * dinv + b_ref[...]
    m = jnp.max(t, axis=1, keepdims=True)
    lse = m + jnp.log(jnp.sum(jnp.exp(t - m), axis=1, keepdims=True))
    o_ref[...] = t - lse


_row_spec = pl.BlockSpec((ROWB, D), lambda i: (i, 0))
_agg_spec = pl.BlockSpec((NC, ROWB, D), lambda i: (0, i, 0))
_deg_spec = pl.BlockSpec((NC, ROWB, DEGL), lambda i: (0, i, 0))
_w_spec = pl.BlockSpec((D, D), lambda i: (0, 0))
_b_spec = pl.BlockSpec((1, D), lambda i: (0, 0))
_out_f32 = jax.ShapeDtypeStruct((N, D), jnp.float32)


def _tc_pre(x, w1, degp):
    return pl.pallas_call(
        _tc_pre_body,
        grid=(N // ROWB,),
        in_specs=[_row_spec, _w_spec, _deg_spec],
        out_specs=_row_spec,
        out_shape=_out_f32,
    )(x, w1, degp)


def _tc_mid(agg1, hs1, degp, b1, w2):
    return pl.pallas_call(
        _tc_mid_body,
        grid=(N // ROWB,),
        in_specs=[_agg_spec, _row_spec, _deg_spec, _b_spec, _w_spec],
        out_specs=_row_spec,
        out_shape=_out_f32,
    )(agg1, hs1, degp, b1, w2)


def _tc_post(agg2, hs2, degp, b2):
    return pl.pallas_call(
        _tc_post_body,
        grid=(N // ROWB,),
        in_specs=[_agg_spec, _row_spec, _deg_spec, _b_spec],
        out_specs=_row_spec,
        out_shape=_out_f32,
    )(agg2, hs2, degp, b2)


def kernel(x, adj_t, W1, b1, W2, b2):
    src = adj_t[0].astype(jnp.int32)
    dst = adj_t[1].astype(jnp.int32)
    pad = E_PAD - E
    srcp = jnp.concatenate([src, jnp.zeros((pad,), jnp.int32)])
    dstp = jnp.concatenate([dst, jnp.full((pad,), N, jnp.int32)])

    degp = _sc_degree(dstp)
    hs1 = _tc_pre(x, W1, degp)
    agg1 = _sc_aggregate(hs1, srcp, dstp)
    hs2 = _tc_mid(agg1, hs1, degp, b1.reshape(1, D), W2)
    agg2 = _sc_aggregate(hs2, srcp, dstp)
    return _tc_post(agg2, hs2, degp, b2.reshape(1, D))


# speedup vs baseline: 10.4641x; 10.4641x over previous
import functools

import jax
import jax.numpy as jnp
from jax import lax
from jax.experimental import pallas as pl
from jax.experimental.pallas import tpu as pltpu
from jax.experimental.pallas import tpu_sc as plsc

N = 10000
E = 320000
D = 128

NC = 2
NS = 16
NW = NC * NS
CHUNK = 128
CPW = 79
EPW = CHUNK * CPW
E_PAD = EPW * NW
N_PAD = 10240
RPS = N_PAD // NS
DEGL = 128

ROWB = 1000

_mesh = plsc.VectorSubcoreMesh(core_axis_name="c", subcore_axis_name="s",
                               num_cores=NC, num_subcores=NS)


@functools.partial(
    pl.kernel,
    out_type=jax.ShapeDtypeStruct((NC, N_PAD, DEGL), jnp.float32),
    mesh=_mesh,
    scratch_types=[
        pltpu.VMEM((CHUNK,), jnp.int32),
        pltpu.VMEM((CHUNK, DEGL), jnp.float32),
        pltpu.VMEM_SHARED((N_PAD, DEGL), jnp.float32),
    ],
)
def _sc_degree(dst_hbm, out_hbm, didx, buf, acc):
    cid = lax.axis_index("c")
    sid = lax.axis_index("s")
    wid = cid * NS + sid

    @pl.loop(0, CHUNK)
    def _(i):
        @pl.loop(0, DEGL // 16)
        def _(j):
            buf[i, pl.ds(j * 16, 16)] = jnp.zeros((16,), jnp.float32)

    row0 = sid * RPS

    @pl.loop(0, RPS // CHUNK)
    def _(b):
        pltpu.sync_copy(buf, acc.at[pl.ds(row0 + b * CHUNK, CHUNK)])

    @pl.loop(0, CHUNK)
    def _(i):
        @pl.loop(0, DEGL // 16)
        def _(j):
            buf[i, pl.ds(j * 16, 16)] = jnp.ones((16,), jnp.float32)

    plsc.subcore_barrier()

    base0 = wid * EPW

    @pl.loop(0, CPW)
    def _(c):
        pltpu.sync_copy(dst_hbm.at[pl.ds(base0 + c * CHUNK, CHUNK)], didx)
        pltpu.sync_copy(buf, acc.at[didx], add=True)

    plsc.subcore_barrier()
    pltpu.sync_copy(acc.at[pl.ds(row0, RPS)], out_hbm.at[cid, pl.ds(row0, RPS)])


@functools.partial(
    pl.kernel,
    out_type=jax.ShapeDtypeStruct((NC, N_PAD, D), jnp.float32),
    mesh=_mesh,
    scratch_types=[
        pltpu.VMEM((CHUNK,), jnp.int32),
        pltpu.VMEM((CHUNK,), jnp.int32),
        pltpu.VMEM((CHUNK, D), jnp.float32),
        pltpu.VMEM_SHARED((N_PAD, D), jnp.float32),
        pltpu.SemaphoreType.DMA,
    ],
)
def _sc_aggregate(table_hbm, src_hbm, dst_hbm, out_hbm, sidx, didx, rows, acc, sem):
    cid = lax.axis_index("c")
    sid = lax.axis_index("s")
    wid = cid * NS + sid

    @pl.loop(0, CHUNK)
    def _(i):
        @pl.loop(0, D // 16)
        def _(j):
            rows[i, pl.ds(j * 16, 16)] = jnp.zeros((16,), jnp.float32)

    row0 = sid * RPS

    @pl.loop(0, RPS // CHUNK)
    def _(b):
        pltpu.sync_copy(rows, acc.at[pl.ds(row0 + b * CHUNK, CHUNK)])

    plsc.subcore_barrier()

    base0 = wid * EPW

    @pl.loop(0, CPW)
    def _(c):
        b = base0 + c * CHUNK
        pltpu.sync_copy(src_hbm.at[pl.ds(b, CHUNK)], sidx)
        pltpu.sync_copy(dst_hbm.at[pl.ds(b, CHUNK)], didx)
        pltpu.async_copy(table_hbm.at[sidx], rows, sem).wait()
        pltpu.sync_copy(rows, acc.at[didx], add=True)

    plsc.subcore_barrier()
    pltpu.sync_copy(acc.at[pl.ds(row0, RPS)], out_hbm.at[cid, pl.ds(row0, RPS)])


def _dinv_block(deg_ref):
    deg = deg_ref[0, :, 0:1] + deg_ref[1, :, 0:1] + 1.0
    return lax.rsqrt(deg)


def _tc_pre_body(x_ref, w_ref, deg_ref, o_ref):
    h = jnp.dot(x_ref[...], w_ref[...], preferred_element_type=jnp.float32)
    o_ref[...] = h * _dinv_block(deg_ref)


def _tc_mid_body(agg_ref, hs_ref, deg_ref, b_ref, w_ref, o_ref):
    dinv = _dinv_block(deg_ref)
    t = (agg_ref[0] + agg_ref[1] + hs_ref[...]) * dinv + b_ref[...]
    r = jnp.maximum(t, 0.0)
    o_ref[...] = jnp.dot(r, w_ref[...], preferred_element_type=jnp.float32) * dinv


def _tc_post_body(agg_ref, hs_ref, deg_ref, b_ref, o_ref):
    dinv = _dinv_block(deg_ref)
    t = (agg_ref[0] + agg_ref[1] + hs_ref[...]) * dinv + b_ref[...]
    m = jnp.max(t, axis=1, keepdims=True)
    lse = m + jnp.log(jnp.sum(jnp.exp(t - m), axis=1, keepdims=True))
    o_ref[...] = t - lse


_row_spec = pl.BlockSpec((ROWB, D), lambda i: (i, 0))
_agg_spec = pl.BlockSpec((NC, ROWB, D), lambda i: (0, i, 0))
_deg_spec = pl.BlockSpec((NC, ROWB, DEGL), lambda i: (0, i, 0))
_w_spec = pl.BlockSpec((D, D), lambda i: (0, 0))
_b_spec = pl.BlockSpec((1, D), lambda i: (0, 0))
_out_f32 = jax.ShapeDtypeStruct((N, D), jnp.float32)


def _tc_pre(x, w1, degp):
    return pl.pallas_call(
        _tc_pre_body,
        grid=(N // ROWB,),
        in_specs=[_row_spec, _w_spec, _deg_spec],
        out_specs=_row_spec,
        out_shape=_out_f32,
    )(x, w1, degp)


def _tc_mid(agg1, hs1, degp, b1, w2):
    return pl.pallas_call(
        _tc_mid_body,
        grid=(N // ROWB,),
        in_specs=[_agg_spec, _row_spec, _deg_spec, _b_spec, _w_spec],
        out_specs=_row_spec,
        out_shape=_out_f32,
    )(agg1, hs1, degp, b1, w2)


def _tc_post(agg2, hs2, degp, b2):
    return pl.pallas_call(
        _tc_post_body,
        grid=(N // ROWB,),
        in_specs=[_agg_spec, _row_spec, _deg_spec, _b_spec],
        out_specs=_row_spec,
        out_shape=_out_f32,
    )(agg2, hs2, degp, b2)


def kernel(x, adj_t, W1, b1, W2, b2):
    src = adj_t[0].astype(jnp.int32)
    dst = adj_t[1].astype(jnp.int32)
    pad = E_PAD - E
    srcp = jnp.concatenate([src, jnp.zeros((pad,), jnp.int32)])
    dstp = jnp.concatenate([dst, jnp.full((pad,), N, jnp.int32)])

    degp = _sc_degree(dstp)
    hs1 = _tc_pre(x, W1, degp)
    agg1 = _sc_aggregate(hs1, srcp, dstp)
    hs2 = _tc_mid(agg1, hs1, degp, b1.reshape(1, D), W2)
    agg2 = _sc_aggregate(hs2, srcp, dstp)
    return _tc_post(agg2, hs2, degp, b2.reshape(1, D))
